# Initial kernel scaffold; baseline (speedup 1.0000x reference)
import jax
import jax.numpy as jnp
from jax.experimental import pallas as pl


def kernel(x, edge_index, W1, att_src1, att_dst1, b1, W2, att_src2, att_dst2, b2):
    raise NotImplementedError("write your pallas kernel here")


# speedup vs baseline: 1.0205x; 1.0205x over previous
import jax
import jax.numpy as jnp
from jax.experimental import pallas as pl

_BLK = 512


def _mm_att_body(x_ref, w_ref, as_w_ref, ad_w_ref, xp_ref, as_ref, ad_ref):
    xp = jnp.dot(x_ref[...], w_ref[...], preferred_element_type=jnp.float32)
    xp_ref[...] = xp
    as_ref[...] = jnp.dot(xp, as_w_ref[...], preferred_element_type=jnp.float32)
    ad_ref[...] = jnp.dot(xp, ad_w_ref[...], preferred_element_type=jnp.float32)


def _proj_att(xpad, W, As, Ad):
    np_, d_in = xpad.shape
    d_out = W.shape[1]
    h = As.shape[1]
    grid = np_ // _BLK
    return pl.pallas_call(
        _mm_att_body,
        grid=(grid,),
        in_specs=[
            pl.BlockSpec((_BLK, d_in), lambda i: (i, 0)),
            pl.BlockSpec((d_in, d_out), lambda i: (0, 0)),
            pl.BlockSpec((d_out, h), lambda i: (0, 0)),
            pl.BlockSpec((d_out, h), lambda i: (0, 0)),
        ],
        out_specs=[
            pl.BlockSpec((_BLK, d_out), lambda i: (i, 0)),
            pl.BlockSpec((_BLK, h), lambda i: (i, 0)),
            pl.BlockSpec((_BLK, h), lambda i: (i, 0)),
        ],
        out_shape=[
            jax.ShapeDtypeStruct((np_, d_out), jnp.float32),
            jax.ShapeDtypeStruct((np_, h), jnp.float32),
            jax.ShapeDtypeStruct((np_, h), jnp.float32),
        ],
    )(xpad, W, As, Ad)


def _relu_mm_att_body(o_ref, b_ref, w_ref, as_w_ref, ad_w_ref,
                      xp_ref, as_ref, ad_ref):
    h = jnp.maximum(o_ref[...] + b_ref[...], 0.0)
    xp = jnp.dot(h, w_ref[...], preferred_element_type=jnp.float32)
    xp_ref[...] = xp
    as_ref[...] = jnp.dot(xp, as_w_ref[...], preferred_element_type=jnp.float32)
    ad_ref[...] = jnp.dot(xp, ad_w_ref[...], preferred_element_type=jnp.float32)


def _relu_proj_att(opad, b, W, As, Ad):
    np_, d_in = opad.shape
    d_out = W.shape[1]
    h = As.shape[1]
    grid = np_ // _BLK
    return pl.pallas_call(
        _relu_mm_att_body,
        grid=(grid,),
        in_specs=[
            pl.BlockSpec((_BLK, d_in), lambda i: (i, 0)),
            pl.BlockSpec((1, d_in), lambda i: (0, 0)),
            pl.BlockSpec((d_in, d_out), lambda i: (0, 0)),
            pl.BlockSpec((d_out, h), lambda i: (0, 0)),
            pl.BlockSpec((d_out, h), lambda i: (0, 0)),
        ],
        out_specs=[
            pl.BlockSpec((_BLK, d_out), lambda i: (i, 0)),
            pl.BlockSpec((_BLK, h), lambda i: (i, 0)),
            pl.BlockSpec((_BLK, h), lambda i: (i, 0)),
        ],
        out_shape=[
            jax.ShapeDtypeStruct((np_, d_out), jnp.float32),
            jax.ShapeDtypeStruct((np_, h), jnp.float32),
            jax.ShapeDtypeStruct((np_, h), jnp.float32),
        ],
    )(opad, b, W, As, Ad)


def _logsoftmax_body(o_ref, b_ref, out_ref):
    y = o_ref[...] + b_ref[...]
    col = jax.lax.broadcasted_iota(jnp.int32, y.shape, 1)
    mask = col < 41
    ym = jnp.where(mask, y, -1e30)
    mx = jnp.max(ym, axis=1, keepdims=True)
    e = jnp.where(mask, jnp.exp(y - mx), 0.0)
    s = jnp.sum(e, axis=1, keepdims=True)
    out_ref[...] = y - mx - jnp.log(s)


def _bias_logsoftmax(opad, b):
    np_, d = opad.shape
    grid = np_ // _BLK
    return pl.pallas_call(
        _logsoftmax_body,
        grid=(grid,),
        in_specs=[
            pl.BlockSpec((_BLK, d), lambda i: (i, 0)),
            pl.BlockSpec((1, d), lambda i: (0, 0)),
        ],
        out_specs=pl.BlockSpec((_BLK, d), lambda i: (i, 0)),
        out_shape=jax.ShapeDtypeStruct((np_, d), jnp.float32),
    )(opad, b)


def _edge_softmax_scatter(xp, a_s, a_d, src, dst, n, heads, ch):
    alpha = a_s[src] + a_d[dst]
    alpha = jax.nn.leaky_relu(alpha, negative_slope=0.2)
    amax = jax.ops.segment_max(alpha, dst, num_segments=n)
    amax = jnp.where(jnp.isfinite(amax), amax, 0.0)
    ex = jnp.exp(alpha - amax[dst])
    denom = jax.ops.segment_sum(ex, dst, num_segments=n)
    coef = ex / (denom[dst] + 1e-16)
    msg = xp.reshape(n, heads, ch)[src] * coef[:, :, None]
    return jax.ops.segment_sum(msg, dst, num_segments=n).reshape(n, heads * ch)


def kernel(x, edge_index, W1, att_src1, att_dst1, b1, W2, att_src2, att_dst2, b2):
    n = x.shape[0]
    heads, hid = att_src1.shape
    out_c = W2.shape[1]
    npad = ((n + _BLK - 1) // _BLK) * _BLK

    loop = jnp.arange(n, dtype=edge_index.dtype)
    ei = jnp.concatenate([edge_index, jnp.stack([loop, loop])], axis=1)
    src, dst = ei[0], ei[1]

    eye = jnp.eye(heads, dtype=jnp.float32)
    As1 = (eye[:, None, :] * att_src1[:, :, None]).reshape(heads * hid, heads)
    Ad1 = (eye[:, None, :] * att_dst1[:, :, None]).reshape(heads * hid, heads)

    xpad = jnp.pad(x, ((0, npad - n), (0, 0)))
    xp1, a_s1, a_d1 = _proj_att(xpad, W1, As1, Ad1)
    xp1, a_s1, a_d1 = xp1[:n], a_s1[:n], a_d1[:n]

    out1 = _edge_softmax_scatter(xp1, a_s1, a_d1, src, dst, n, heads, hid)

    dpad = 128
    W2p = jnp.pad(W2, ((0, 0), (0, dpad - out_c)))
    As2 = jnp.zeros((dpad, 8), jnp.float32).at[:out_c, 0].set(att_src2[0])
    Ad2 = jnp.zeros((dpad, 8), jnp.float32).at[:out_c, 0].set(att_dst2[0])

    o1pad = jnp.pad(out1, ((0, npad - n), (0, 0)))
    xp2, a_s2, a_d2 = _relu_proj_att(o1pad, b1.reshape(1, -1), W2p, As2, Ad2)
    xp2 = xp2[:n, :out_c]
    a_s2, a_d2 = a_s2[:n, :1], a_d2[:n, :1]

    out2 = _edge_softmax_scatter(xp2, a_s2, a_d2, src, dst, n, 1, out_c)

    o2pad = jnp.pad(out2, ((0, npad - n), (0, dpad - out_c)))
    b2p = jnp.pad(b2, (0, dpad - out_c)).reshape(1, dpad)
    res = _bias_logsoftmax(o2pad, b2p)
    return res[:n, :out_c]


# speedup vs baseline: 2.5697x; 2.5181x over previous
import functools

import jax
import jax.numpy as jnp
from jax import lax
from jax.experimental import pallas as pl
from jax.experimental.pallas import tpu as pltpu, tpu_sc as plsc

_BLK = 512
_N = 10000
_H = 8
_C = 128
_NW = 32
_EC = 10320
_EPAD = _EC * _NW
_RPW = 632
_NACC = _RPW * 16


def _sc_spmm(xp_flat, coef_flat, src_pad, dst_pad, zeros_rows):
    mesh = plsc.VectorSubcoreMesh(core_axis_name="c", subcore_axis_name="s")

    @functools.partial(
        pl.kernel, mesh=mesh,
        out_type=jax.ShapeDtypeStruct((2 * _H * _NACC, _C), jnp.float32),
        scratch_types=[
            pltpu.VMEM((_EC,), jnp.int32),
            pltpu.VMEM((_EC,), jnp.int32),
            pltpu.VMEM((_EC,), jnp.float32),
            pltpu.VMEM((16, _C), jnp.float32),
            pltpu.VMEM_SHARED((_NACC, _C), jnp.float32),
            pltpu.SemaphoreType.DMA,
        ],
    )
    def k(xp_hbm, coef_hbm, src_hbm, dst_hbm, z_hbm, out_hbm,
          src_v, dst_v, coef_v, rows_v, acc, sem):
        core = lax.axis_index("c")
        sid = lax.axis_index("s")
        wid = sid * 2 + core
        base = wid * _EC
        r0 = sid * _RPW
        iota = lax.iota(jnp.int32, 16)

        pltpu.sync_copy(src_hbm.at[pl.ds(base, _EC)], src_v)
        pltpu.sync_copy(dst_hbm.at[pl.ds(base, _EC)], dst_v)

        for h in range(_H):
            pltpu.sync_copy(coef_hbm.at[pl.ds(h * _EPAD + base, _EC)], coef_v)
            pltpu.sync_copy(z_hbm, acc.at[pl.ds(r0, _RPW)])
            plsc.subcore_barrier()

            def chunk(i, carry):
                off = i * 16
                s16 = src_v[pl.ds(off, 16)]
                d16 = dst_v[pl.ds(off, 16)]
                cvec = coef_v[pl.ds(off, 16)]
                pltpu.async_copy(xp_hbm.at[s16 + h * _N], rows_v, sem).wait()
                for j in range(16):
                    cb = cvec[j]
                    for g in range(8):
                        sl = pl.ds(g * 16, 16)
                        rows_v[j, sl] = rows_v[j, sl] * cb
                pltpu.sync_copy(rows_v, acc.at[d16], add=True)
                return carry

            lax.fori_loop(0, _EC // 16, chunk, 0)
            plsc.subcore_barrier()
            out_off = (core * _H + h) * _NACC + r0
            pltpu.sync_copy(acc.at[pl.ds(r0, _RPW)],
                            out_hbm.at[pl.ds(out_off, _RPW)])
            plsc.subcore_barrier()

    return k(xp_flat, coef_flat, src_pad, dst_pad, zeros_rows)


def _mm_att_body(x_ref, w_ref, as_w_ref, ad_w_ref, xp_ref, as_ref, ad_ref):
    xp = jnp.dot(x_ref[...], w_ref[...], preferred_element_type=jnp.float32)
    xp_ref[...] = xp
    as_ref[...] = jnp.dot(xp, as_w_ref[...], preferred_element_type=jnp.float32)
    ad_ref[...] = jnp.dot(xp, ad_w_ref[...], preferred_element_type=jnp.float32)


def _proj_att(xpad, W, As, Ad):
    np_, d_in = xpad.shape
    d_out = W.shape[1]
    h = As.shape[1]
    grid = np_ // _BLK
    return pl.pallas_call(
        _mm_att_body,
        grid=(grid,),
        in_specs=[
            pl.BlockSpec((_BLK, d_in), lambda i: (i, 0)),
            pl.BlockSpec((d_in, d_out), lambda i: (0, 0)),
            pl.BlockSpec((d_out, h), lambda i: (0, 0)),
            pl.BlockSpec((d_out, h), lambda i: (0, 0)),
        ],
        out_specs=[
            pl.BlockSpec((_BLK, d_out), lambda i: (i, 0)),
            pl.BlockSpec((_BLK, h), lambda i: (i, 0)),
            pl.BlockSpec((_BLK, h), lambda i: (i, 0)),
        ],
        out_shape=[
            jax.ShapeDtypeStruct((np_, d_out), jnp.float32),
            jax.ShapeDtypeStruct((np_, h), jnp.float32),
            jax.ShapeDtypeStruct((np_, h), jnp.float32),
        ],
    )(xpad, W, As, Ad)


def _relu_mm_att_body(o_ref, b_ref, w_ref, as_w_ref, ad_w_ref,
                      xp_ref, as_ref, ad_ref):
    h = jnp.maximum(o_ref[...] + b_ref[...], 0.0)
    xp = jnp.dot(h, w_ref[...], preferred_element_type=jnp.float32)
    xp_ref[...] = xp
    as_ref[...] = jnp.dot(xp, as_w_ref[...], preferred_element_type=jnp.float32)
    ad_ref[...] = jnp.dot(xp, ad_w_ref[...], preferred_element_type=jnp.float32)


def _relu_proj_att(opad, b, W, As, Ad):
    np_, d_in = opad.shape
    d_out = W.shape[1]
    h = As.shape[1]
    grid = np_ // _BLK
    return pl.pallas_call(
        _relu_mm_att_body,
        grid=(grid,),
        in_specs=[
            pl.BlockSpec((_BLK, d_in), lambda i: (i, 0)),
            pl.BlockSpec((1, d_in), lambda i: (0, 0)),
            pl.BlockSpec((d_in, d_out), lambda i: (0, 0)),
            pl.BlockSpec((d_out, h), lambda i: (0, 0)),
            pl.BlockSpec((d_out, h), lambda i: (0, 0)),
        ],
        out_specs=[
            pl.BlockSpec((_BLK, d_out), lambda i: (i, 0)),
            pl.BlockSpec((_BLK, h), lambda i: (i, 0)),
            pl.BlockSpec((_BLK, h), lambda i: (i, 0)),
        ],
        out_shape=[
            jax.ShapeDtypeStruct((np_, d_out), jnp.float32),
            jax.ShapeDtypeStruct((np_, h), jnp.float32),
            jax.ShapeDtypeStruct((np_, h), jnp.float32),
        ],
    )(opad, b, W, As, Ad)


def _logsoftmax_body(o_ref, b_ref, out_ref):
    y = o_ref[...] + b_ref[...]
    col = jax.lax.broadcasted_iota(jnp.int32, y.shape, 1)
    mask = col < 41
    ym = jnp.where(mask, y, -1e30)
    mx = jnp.max(ym, axis=1, keepdims=True)
    e = jnp.where(mask, jnp.exp(y - mx), 0.0)
    s = jnp.sum(e, axis=1, keepdims=True)
    out_ref[...] = y - mx - jnp.log(s)


def _bias_logsoftmax(opad, b):
    np_, d = opad.shape
    grid = np_ // _BLK
    return pl.pallas_call(
        _logsoftmax_body,
        grid=(grid,),
        in_specs=[
            pl.BlockSpec((_BLK, d), lambda i: (i, 0)),
            pl.BlockSpec((1, d), lambda i: (0, 0)),
        ],
        out_specs=pl.BlockSpec((_BLK, d), lambda i: (i, 0)),
        out_shape=jax.ShapeDtypeStruct((np_, d), jnp.float32),
    )(opad, b)


def _edge_softmax_scatter(xp, a_s, a_d, src, dst, n, heads, ch):
    alpha = a_s[src] + a_d[dst]
    alpha = jax.nn.leaky_relu(alpha, negative_slope=0.2)
    amax = jax.ops.segment_max(alpha, dst, num_segments=n)
    amax = jnp.where(jnp.isfinite(amax), amax, 0.0)
    ex = jnp.exp(alpha - amax[dst])
    denom = jax.ops.segment_sum(ex, dst, num_segments=n)
    coef = ex / (denom[dst] + 1e-16)
    msg = xp.reshape(n, heads, ch)[src] * coef[:, :, None]
    return jax.ops.segment_sum(msg, dst, num_segments=n).reshape(n, heads * ch)


def kernel(x, edge_index, W1, att_src1, att_dst1, b1, W2, att_src2, att_dst2, b2):
    n = x.shape[0]
    heads, hid = att_src1.shape
    out_c = W2.shape[1]
    npad = ((n + _BLK - 1) // _BLK) * _BLK

    loop = jnp.arange(n, dtype=edge_index.dtype)
    ei = jnp.concatenate([edge_index, jnp.stack([loop, loop])], axis=1)
    src, dst = ei[0], ei[1]

    eye = jnp.eye(heads, dtype=jnp.float32)
    As1 = (eye[:, None, :] * att_src1[:, :, None]).reshape(heads * hid, heads)
    Ad1 = (eye[:, None, :] * att_dst1[:, :, None]).reshape(heads * hid, heads)

    xpad = jnp.pad(x, ((0, npad - n), (0, 0)))
    xp1, a_s1, a_d1 = _proj_att(xpad, W1, As1, Ad1)
    xp1, a_s1, a_d1 = xp1[:n], a_s1[:n], a_d1[:n]

    alpha = a_s1[src] + a_d1[dst]
    alpha = jax.nn.leaky_relu(alpha, negative_slope=0.2)
    amax = jax.ops.segment_max(alpha, dst, num_segments=n)
    amax = jnp.where(jnp.isfinite(amax), amax, 0.0)
    ex = jnp.exp(alpha - amax[dst])
    denom = jax.ops.segment_sum(ex, dst, num_segments=n)
    coef = ex / (denom[dst] + 1e-16)

    e_real = src.shape[0]
    coef_flat = jnp.pad(coef.T, ((0, 0), (0, _EPAD - e_real))).reshape(-1)
    src_pad = jnp.pad(src.astype(jnp.int32), (0, _EPAD - e_real))
    dst_pad = jnp.pad(dst.astype(jnp.int32), (0, _EPAD - e_real))
    xp_flat = xp1.reshape(n, heads, hid).transpose(1, 0, 2).reshape(-1, hid)
    zeros_rows = jnp.zeros((_RPW, hid), jnp.float32)
    out_part = _sc_spmm(xp_flat, coef_flat, src_pad, dst_pad, zeros_rows)
    out1 = out_part.reshape(2, heads, _NACC, hid)[:, :, :n].sum(0)
    out1 = out1.transpose(1, 0, 2).reshape(n, heads * hid)

    dpad = 128
    W2p = jnp.pad(W2, ((0, 0), (0, dpad - out_c)))
    As2 = jnp.zeros((dpad, 8), jnp.float32).at[:out_c, 0].set(att_src2[0])
    Ad2 = jnp.zeros((dpad, 8), jnp.float32).at[:out_c, 0].set(att_dst2[0])

    o1pad = jnp.pad(out1, ((0, npad - n), (0, 0)))
    xp2, a_s2, a_d2 = _relu_proj_att(o1pad, b1.reshape(1, -1), W2p, As2, Ad2)
    xp2 = xp2[:n, :out_c]
    a_s2, a_d2 = a_s2[:n, :1], a_d2[:n, :1]

    out2 = _edge_softmax_scatter(xp2, a_s2, a_d2, src, dst, n, 1, out_c)

    o2pad = jnp.pad(out2, ((0, npad - n), (0, dpad - out_c)))
    b2p = jnp.pad(b2, (0, dpad - out_c)).reshape(1, dpad)
    res = _bias_logsoftmax(o2pad, b2p)
    return res[:n, :out_c]


# speedup vs baseline: 2.7705x; 1.0781x over previous
import functools

import jax
import jax.numpy as jnp
from jax import lax
from jax.experimental import pallas as pl
from jax.experimental.pallas import tpu as pltpu, tpu_sc as plsc

_BLK = 512
_N = 10000
_H = 8
_C = 128
_NW = 32
_EC = 10320
_EPAD = _EC * _NW
_RPW = 632
_NACC = _RPW * 16


def _sc_spmm(xp_flat, coef_flat, src_pad, dst_pad, zeros_rows, nh, ch):
    mesh = plsc.VectorSubcoreMesh(core_axis_name="c", subcore_axis_name="s")

    @functools.partial(
        pl.kernel, mesh=mesh,
        out_type=jax.ShapeDtypeStruct((2 * nh * _NACC, ch), jnp.float32),
        scratch_types=[
            pltpu.VMEM((_EC,), jnp.int32),
            pltpu.VMEM((_EC,), jnp.int32),
            pltpu.VMEM((_EC,), jnp.float32),
            pltpu.VMEM((16, ch), jnp.float32),
            pltpu.VMEM_SHARED((_NACC, ch), jnp.float32),
            pltpu.SemaphoreType.DMA,
        ],
    )
    def k(xp_hbm, coef_hbm, src_hbm, dst_hbm, z_hbm, out_hbm,
          src_v, dst_v, coef_v, rows_v, acc, sem):
        core = lax.axis_index("c")
        sid = lax.axis_index("s")
        wid = sid * 2 + core
        base = wid * _EC
        r0 = sid * _RPW
        iota = lax.iota(jnp.int32, 16)

        pltpu.sync_copy(src_hbm.at[pl.ds(base, _EC)], src_v)
        pltpu.sync_copy(dst_hbm.at[pl.ds(base, _EC)], dst_v)

        for h in range(nh):
            pltpu.sync_copy(coef_hbm.at[pl.ds(h * _EPAD + base, _EC)], coef_v)
            pltpu.sync_copy(z_hbm, acc.at[pl.ds(r0, _RPW)])
            plsc.subcore_barrier()

            def chunk(i, carry):
                off = i * 16
                s16 = src_v[pl.ds(off, 16)]
                d16 = dst_v[pl.ds(off, 16)]
                cvec = coef_v[pl.ds(off, 16)]
                pltpu.async_copy(xp_hbm.at[s16 + h * _N], rows_v, sem).wait()
                for j in range(16):
                    cb = cvec[j]
                    for g in range(ch // 16):
                        sl = pl.ds(g * 16, 16)
                        rows_v[j, sl] = rows_v[j, sl] * cb
                pltpu.sync_copy(rows_v, acc.at[d16], add=True)
                return carry

            lax.fori_loop(0, _EC // 16, chunk, 0)
            plsc.subcore_barrier()
            out_off = (core * nh + h) * _NACC + r0
            pltpu.sync_copy(acc.at[pl.ds(r0, _RPW)],
                            out_hbm.at[pl.ds(out_off, _RPW)])
            plsc.subcore_barrier()

    return k(xp_flat, coef_flat, src_pad, dst_pad, zeros_rows)


def _mm_att_body(x_ref, w_ref, as_w_ref, ad_w_ref, xp_ref, as_ref, ad_ref):
    xp = jnp.dot(x_ref[...], w_ref[...], preferred_element_type=jnp.float32)
    xp_ref[...] = xp
    as_ref[...] = jnp.dot(xp, as_w_ref[...], preferred_element_type=jnp.float32)
    ad_ref[...] = jnp.dot(xp, ad_w_ref[...], preferred_element_type=jnp.float32)


def _proj_att(xpad, W, As, Ad):
    np_, d_in = xpad.shape
    d_out = W.shape[1]
    h = As.shape[1]
    grid = np_ // _BLK
    return pl.pallas_call(
        _mm_att_body,
        grid=(grid,),
        in_specs=[
            pl.BlockSpec((_BLK, d_in), lambda i: (i, 0)),
            pl.BlockSpec((d_in, d_out), lambda i: (0, 0)),
            pl.BlockSpec((d_out, h), lambda i: (0, 0)),
            pl.BlockSpec((d_out, h), lambda i: (0, 0)),
        ],
        out_specs=[
            pl.BlockSpec((_BLK, d_out), lambda i: (i, 0)),
            pl.BlockSpec((_BLK, h), lambda i: (i, 0)),
            pl.BlockSpec((_BLK, h), lambda i: (i, 0)),
        ],
        out_shape=[
            jax.ShapeDtypeStruct((np_, d_out), jnp.float32),
            jax.ShapeDtypeStruct((np_, h), jnp.float32),
            jax.ShapeDtypeStruct((np_, h), jnp.float32),
        ],
    )(xpad, W, As, Ad)


def _relu_mm_att_body(o_ref, b_ref, w_ref, as_w_ref, ad_w_ref,
                      xp_ref, as_ref, ad_ref):
    h = jnp.maximum(o_ref[...] + b_ref[...], 0.0)
    xp = jnp.dot(h, w_ref[...], preferred_element_type=jnp.float32)
    xp_ref[...] = xp
    as_ref[...] = jnp.dot(xp, as_w_ref[...], preferred_element_type=jnp.float32)
    ad_ref[...] = jnp.dot(xp, ad_w_ref[...], preferred_element_type=jnp.float32)


def _relu_proj_att(opad, b, W, As, Ad):
    np_, d_in = opad.shape
    d_out = W.shape[1]
    h = As.shape[1]
    grid = np_ // _BLK
    return pl.pallas_call(
        _relu_mm_att_body,
        grid=(grid,),
        in_specs=[
            pl.BlockSpec((_BLK, d_in), lambda i: (i, 0)),
            pl.BlockSpec((1, d_in), lambda i: (0, 0)),
            pl.BlockSpec((d_in, d_out), lambda i: (0, 0)),
            pl.BlockSpec((d_out, h), lambda i: (0, 0)),
            pl.BlockSpec((d_out, h), lambda i: (0, 0)),
        ],
        out_specs=[
            pl.BlockSpec((_BLK, d_out), lambda i: (i, 0)),
            pl.BlockSpec((_BLK, h), lambda i: (i, 0)),
            pl.BlockSpec((_BLK, h), lambda i: (i, 0)),
        ],
        out_shape=[
            jax.ShapeDtypeStruct((np_, d_out), jnp.float32),
            jax.ShapeDtypeStruct((np_, h), jnp.float32),
            jax.ShapeDtypeStruct((np_, h), jnp.float32),
        ],
    )(opad, b, W, As, Ad)


def _logsoftmax_body(o_ref, b_ref, out_ref):
    y = o_ref[...] + b_ref[...]
    col = jax.lax.broadcasted_iota(jnp.int32, y.shape, 1)
    mask = col < 41
    ym = jnp.where(mask, y, -1e30)
    mx = jnp.max(ym, axis=1, keepdims=True)
    e = jnp.where(mask, jnp.exp(y - mx), 0.0)
    s = jnp.sum(e, axis=1, keepdims=True)
    out_ref[...] = y - mx - jnp.log(s)


def _bias_logsoftmax(opad, b):
    np_, d = opad.shape
    grid = np_ // _BLK
    return pl.pallas_call(
        _logsoftmax_body,
        grid=(grid,),
        in_specs=[
            pl.BlockSpec((_BLK, d), lambda i: (i, 0)),
            pl.BlockSpec((1, d), lambda i: (0, 0)),
        ],
        out_specs=pl.BlockSpec((_BLK, d), lambda i: (i, 0)),
        out_shape=jax.ShapeDtypeStruct((np_, d), jnp.float32),
    )(opad, b)


def _edge_softmax_scatter(xp, a_s, a_d, src, dst, n, heads, ch):
    alpha = a_s[src] + a_d[dst]
    alpha = jax.nn.leaky_relu(alpha, negative_slope=0.2)
    amax = jax.ops.segment_max(alpha, dst, num_segments=n)
    amax = jnp.where(jnp.isfinite(amax), amax, 0.0)
    ex = jnp.exp(alpha - amax[dst])
    denom = jax.ops.segment_sum(ex, dst, num_segments=n)
    coef = ex / (denom[dst] + 1e-16)
    msg = xp.reshape(n, heads, ch)[src] * coef[:, :, None]
    return jax.ops.segment_sum(msg, dst, num_segments=n).reshape(n, heads * ch)


def kernel(x, edge_index, W1, att_src1, att_dst1, b1, W2, att_src2, att_dst2, b2):
    n = x.shape[0]
    heads, hid = att_src1.shape
    out_c = W2.shape[1]
    npad = ((n + _BLK - 1) // _BLK) * _BLK

    loop = jnp.arange(n, dtype=edge_index.dtype)
    ei = jnp.concatenate([edge_index, jnp.stack([loop, loop])], axis=1)
    src, dst = ei[0], ei[1]

    eye = jnp.eye(heads, dtype=jnp.float32)
    As1 = (eye[:, None, :] * att_src1[:, :, None]).reshape(heads * hid, heads)
    Ad1 = (eye[:, None, :] * att_dst1[:, :, None]).reshape(heads * hid, heads)

    xpad = jnp.pad(x, ((0, npad - n), (0, 0)))
    xp1, a_s1, a_d1 = _proj_att(xpad, W1, As1, Ad1)
    xp1, a_s1, a_d1 = xp1[:n], a_s1[:n], a_d1[:n]

    alpha = a_s1[src] + a_d1[dst]
    alpha = jax.nn.leaky_relu(alpha, negative_slope=0.2)
    amax = jax.ops.segment_max(alpha, dst, num_segments=n)
    amax = jnp.where(jnp.isfinite(amax), amax, 0.0)
    ex = jnp.exp(alpha - amax[dst])
    denom = jax.ops.segment_sum(ex, dst, num_segments=n)
    coef = ex / (denom[dst] + 1e-16)

    e_real = src.shape[0]
    coef_flat = jnp.pad(coef.T, ((0, 0), (0, _EPAD - e_real))).reshape(-1)
    src_pad = jnp.pad(src.astype(jnp.int32), (0, _EPAD - e_real))
    dst_pad = jnp.pad(dst.astype(jnp.int32), (0, _EPAD - e_real))
    xp_flat = xp1.reshape(n, heads, hid).transpose(1, 0, 2).reshape(-1, hid)
    zeros_rows = jnp.zeros((_RPW, hid), jnp.float32)
    out_part = _sc_spmm(xp_flat, coef_flat, src_pad, dst_pad, zeros_rows,
                        heads, hid)
    out1 = out_part.reshape(2, heads, _NACC, hid)[:, :, :n].sum(0)
    out1 = out1.transpose(1, 0, 2).reshape(n, heads * hid)

    dpad = 128
    W2p = jnp.pad(W2, ((0, 0), (0, dpad - out_c)))
    As2 = jnp.zeros((dpad, 8), jnp.float32).at[:out_c, 0].set(att_src2[0])
    Ad2 = jnp.zeros((dpad, 8), jnp.float32).at[:out_c, 0].set(att_dst2[0])

    o1pad = jnp.pad(out1, ((0, npad - n), (0, 0)))
    xp2, a_s2, a_d2 = _relu_proj_att(o1pad, b1.reshape(1, -1), W2p, As2, Ad2)
    xp2 = xp2[:n, :out_c]
    a_s2, a_d2 = a_s2[:n, :1], a_d2[:n, :1]

    alpha2 = jax.nn.leaky_relu(a_s2[src] + a_d2[dst], negative_slope=0.2)
    amax2 = jax.ops.segment_max(alpha2, dst, num_segments=n)
    amax2 = jnp.where(jnp.isfinite(amax2), amax2, 0.0)
    ex2 = jnp.exp(alpha2 - amax2[dst])
    denom2 = jax.ops.segment_sum(ex2, dst, num_segments=n)
    coef2 = ex2 / (denom2[dst] + 1e-16)
    cpad = 128
    xp2p = jnp.pad(xp2, ((0, 0), (0, cpad - out_c)))
    coef2_flat = jnp.pad(coef2.T, ((0, 0), (0, _EPAD - e_real))).reshape(-1)
    zeros2 = jnp.zeros((_RPW, cpad), jnp.float32)
    part2 = _sc_spmm(xp2p, coef2_flat, src_pad, dst_pad, zeros2, 1, cpad)
    out2 = part2.reshape(2, _NACC, cpad)[:, :n, :out_c].sum(0)

    o2pad = jnp.pad(out2, ((0, npad - n), (0, dpad - out_c)))
    b2p = jnp.pad(b2, (0, dpad - out_c)).reshape(1, dpad)
    res = _bias_logsoftmax(o2pad, b2p)
    return res[:n, :out_c]
